# baseline (speedup 1.0000x reference)
import functools

import jax
import jax.numpy as jnp
from jax import lax
from jax.experimental import pallas as pl
from jax.experimental.pallas import tpu as pltpu
from jax.experimental.pallas import tpu_sc as plsc

_B, _N, _S, _D1, _D2 = 16, 4096, 1024, 256, 512
_CIN = _D1 + _D2
_M0, _M1 = 512, 512
_BLK = 512
_NB = _N // _BLK
_BLK2 = 1024
_NB2 = _N // _BLK2

_NW = 32
_PW = (_B * _N) // _NW
_C = 16
_NCHUNK = _PW // _C


def _stage0_body(x2t_ref, x1_ref, idx_ref, w_ref):
    b = pl.program_id(0)
    x2t = x2t_ref[0]
    x1b = x1_ref[0]
    n2 = 0.25 * jnp.sum(x2t * x2t, axis=1, keepdims=True)
    n1 = jnp.sum(x1b * x1b, axis=0, keepdims=True)
    d = jnp.dot(x2t, x1b, preferred_element_type=jnp.float32) + (n2 + n1)

    iota0 = lax.broadcasted_iota(jnp.int32, (_S, _BLK), 0)
    idxs, recs = [], []
    rec_sum = jnp.zeros((1, _BLK), jnp.float32)
    for k in range(3):
        mval = jnp.min(d, axis=0, keepdims=True)
        midx = jnp.min(jnp.where(d == mval, iota0, _S), axis=0, keepdims=True)
        if k < 2:
            d = jnp.where(iota0 == midx, jnp.float32(jnp.inf), d)
        r = 1.0 / (mval + 1e-8)
        idxs.append(midx)
        recs.append(r)
        rec_sum = rec_sum + r
    inv = 1.0 / rec_sum
    idx_ref[0] = jnp.concatenate(idxs, axis=0) + b * _S
    w_ref[0] = jnp.concatenate([r * inv for r in recs], axis=0)


def _sc_gather_body(p2r_hbm, idxp_hbm, wp_hbm, out_hbm,
                    idx_v, w_v, rows_v, acc_v, sem):
    wid = lax.axis_index("s") * 2 + lax.axis_index("c")
    base = wid * _PW

    def chunk(i, carry):
        pb = base + i * _C
        for k in range(3):
            pltpu.sync_copy(idxp_hbm.at[k, pl.ds(pb, _C)],
                            idx_v.at[pl.ds(k * _C, _C)])
            pltpu.sync_copy(wp_hbm.at[k, pl.ds(pb, _C)],
                            w_v.at[pl.ds(k * _C, _C)])
        pltpu.async_copy(p2r_hbm.at[idx_v], rows_v, sem).wait()
        w0all = w_v[pl.ds(0, 16)]
        w1all = w_v[pl.ds(_C, 16)]
        w2all = w_v[pl.ds(2 * _C, 16)]

        def point(p, carry2):
            bidx = jnp.zeros((16,), jnp.int32) + p
            w0 = w0all.at[bidx].get(mode="promise_in_bounds")
            w1 = w1all.at[bidx].get(mode="promise_in_bounds")
            w2 = w2all.at[bidx].get(mode="promise_in_bounds")
            for c in range(_D2 // 16):
                sl = pl.ds(c * 16, 16)
                acc_v[p, sl] = (w0 * rows_v[p, sl]
                                + w1 * rows_v[_C + p, sl]
                                + w2 * rows_v[2 * _C + p, sl])
            return carry2

        lax.fori_loop(0, _C, point, 0)
        pltpu.sync_copy(acc_v, out_hbm.at[pl.ds(pb, _C)])
        return carry

    lax.fori_loop(0, _NCHUNK, chunk, 0)


def _stage1_body(it_ref, p1_ref, w1a_ref, w1b_ref, h_ref, sum_ref, sq_ref):
    interp = it_ref[...]
    h = lax.dot_general(w1a_ref[...], interp.astype(jnp.bfloat16),
                        (((1,), (1,)), ((), ())),
                        preferred_element_type=jnp.float32)
    h = h + jnp.dot(w1b_ref[...], p1_ref[0].astype(jnp.bfloat16),
                    preferred_element_type=jnp.float32)
    h_ref[0] = h.astype(jnp.bfloat16)

    @pl.when((pl.program_id(0) == 0) & (pl.program_id(1) == 0))
    def _init():
        sum_ref[...] = jnp.zeros_like(sum_ref)
        sq_ref[...] = jnp.zeros_like(sq_ref)

    sum_ref[...] += jnp.sum(h, axis=1, keepdims=True)
    sq_ref[...] += jnp.sum(h * h, axis=1, keepdims=True)


def _stage2_body(h_ref, sum_ref, sq_ref, g1_ref, be_ref, w2_ref, b2_ref,
                 out_ref):
    inv_cnt = 1.0 / (_B * _N)
    mean = sum_ref[...] * inv_cnt
    var = sq_ref[...] * inv_cnt - mean * mean
    scale = g1_ref[...] * lax.rsqrt(var + 1e-5)
    shift = be_ref[...] - mean * scale
    g = jnp.maximum(h_ref[0].astype(jnp.float32) * scale + shift, 0.0)
    o = jnp.dot(w2_ref[...], g.astype(jnp.bfloat16),
                preferred_element_type=jnp.float32) + b2_ref[...]
    out_ref[0] = jnp.maximum(o, 0.0)


def kernel(xyz1, xyz2, points1, points2, W1, b1, gamma1, beta1, W2, b2):
    del b1
    x2t = jnp.transpose(xyz2, (0, 2, 1)) * (-2.0)
    w1a = W1[:, :_D2].astype(jnp.bfloat16)
    w1b = W1[:, _D2:].astype(jnp.bfloat16)

    idx3, w3 = pl.pallas_call(
        _stage0_body,
        grid=(_B, _NB),
        in_specs=[
            pl.BlockSpec((1, _S, 3), lambda b, n: (b, 0, 0)),
            pl.BlockSpec((1, 3, _BLK), lambda b, n: (b, 0, n)),
        ],
        out_specs=[
            pl.BlockSpec((1, 3, _BLK), lambda b, n: (b, 0, n)),
            pl.BlockSpec((1, 3, _BLK), lambda b, n: (b, 0, n)),
        ],
        out_shape=[
            jax.ShapeDtypeStruct((_B, 3, _N), jnp.int32),
            jax.ShapeDtypeStruct((_B, 3, _N), jnp.float32),
        ],
    )(x2t, xyz1)

    idxp = jnp.transpose(idx3, (1, 0, 2)).reshape(3, _B * _N)
    wp = jnp.transpose(w3, (1, 0, 2)).reshape(3, _B * _N)
    p2r = jnp.transpose(points2, (0, 2, 1)).reshape(_B * _S, _D2)

    sc_gather = functools.partial(
        pl.kernel,
        mesh=plsc.VectorSubcoreMesh(core_axis_name="c", subcore_axis_name="s"),
        out_type=jax.ShapeDtypeStruct((_B * _N, _D2), jnp.float32),
        scratch_types=[
            pltpu.VMEM((3 * _C,), jnp.int32),
            pltpu.VMEM((3 * _C,), jnp.float32),
            pltpu.VMEM((3 * _C, _D2), jnp.float32),
            pltpu.VMEM((_C, _D2), jnp.float32),
            pltpu.SemaphoreType.DMA,
        ],
    )(_sc_gather_body)
    interp = sc_gather(p2r, idxp, wp)

    h, hsum, hsq = pl.pallas_call(
        _stage1_body,
        grid=(_B, _NB),
        in_specs=[
            pl.BlockSpec((_BLK, _D2), lambda b, n: (b * _NB + n, 0)),
            pl.BlockSpec((1, _D1, _BLK), lambda b, n: (b, 0, n)),
            pl.BlockSpec((_M0, _D2), lambda b, n: (0, 0)),
            pl.BlockSpec((_M0, _D1), lambda b, n: (0, 0)),
        ],
        out_specs=[
            pl.BlockSpec((1, _M0, _BLK), lambda b, n: (b, 0, n)),
            pl.BlockSpec((_M0, 1), lambda b, n: (0, 0)),
            pl.BlockSpec((_M0, 1), lambda b, n: (0, 0)),
        ],
        out_shape=[
            jax.ShapeDtypeStruct((_B, _M0, _N), jnp.bfloat16),
            jax.ShapeDtypeStruct((_M0, 1), jnp.float32),
            jax.ShapeDtypeStruct((_M0, 1), jnp.float32),
        ],
    )(interp, points1, w1a, w1b)

    out = pl.pallas_call(
        _stage2_body,
        grid=(_B, _NB2),
        in_specs=[
            pl.BlockSpec((1, _M0, _BLK2), lambda b, n: (b, 0, n)),
            pl.BlockSpec((_M0, 1), lambda b, n: (0, 0)),
            pl.BlockSpec((_M0, 1), lambda b, n: (0, 0)),
            pl.BlockSpec((_M0, 1), lambda b, n: (0, 0)),
            pl.BlockSpec((_M0, 1), lambda b, n: (0, 0)),
            pl.BlockSpec((_M1, _M0), lambda b, n: (0, 0)),
            pl.BlockSpec((_M1, 1), lambda b, n: (0, 0)),
        ],
        out_specs=pl.BlockSpec((1, _M1, _BLK2), lambda b, n: (b, 0, n)),
        out_shape=jax.ShapeDtypeStruct((_B, _M1, _N), jnp.float32),
    )(h, hsum, hsq, gamma1[:, None], beta1[:, None],
      W2.astype(jnp.bfloat16), b2[:, None])
    return out


# speedup vs baseline: 3.1838x; 3.1838x over previous
import jax
import jax.numpy as jnp
from jax import lax
from jax.experimental import pallas as pl

_B, _N, _S, _D1, _D2 = 16, 4096, 1024, 256, 512
_CIN = _D1 + _D2
_M0, _M1 = 512, 512
_BLK = 1024
_NB = _N // _BLK
_BLK2 = 1024
_NB2 = _N // _BLK2


def _stage1_body(x2t_ref, x1_ref, p2_ref, p1_ref, w1a_ref, w1b_ref,
                 h_ref, sum_ref, sq_ref):
    x2t = x2t_ref[0]
    x1b = x1_ref[0]
    n2 = 0.25 * jnp.sum(x2t * x2t, axis=1, keepdims=True)
    n1 = jnp.sum(x1b * x1b, axis=0, keepdims=True)
    d = jnp.dot(x2t, x1b, preferred_element_type=jnp.float32) + (n2 + n1)

    m1 = jnp.min(d, axis=0, keepdims=True)
    m2 = jnp.min(jnp.where(d <= m1, jnp.float32(jnp.inf), d), axis=0,
                 keepdims=True)
    m3 = jnp.min(jnp.where(d <= m2, jnp.float32(jnp.inf), d), axis=0,
                 keepdims=True)
    inv_rs = 1.0 / (1.0 / (m1 + 1e-8) + 1.0 / (m2 + 1e-8) + 1.0 / (m3 + 1e-8))
    wgt = jnp.where(d <= m3, inv_rs / (d + 1e-8), 0.0).astype(jnp.bfloat16)

    interp = jnp.dot(p2_ref[0].astype(jnp.bfloat16), wgt,
                     preferred_element_type=jnp.float32)
    h = jnp.dot(w1a_ref[...], interp.astype(jnp.bfloat16),
                preferred_element_type=jnp.float32)
    h = h + jnp.dot(w1b_ref[...], p1_ref[0].astype(jnp.bfloat16),
                    preferred_element_type=jnp.float32)
    h_ref[0] = h.astype(jnp.bfloat16)

    @pl.when((pl.program_id(0) == 0) & (pl.program_id(1) == 0))
    def _init():
        sum_ref[...] = jnp.zeros_like(sum_ref)
        sq_ref[...] = jnp.zeros_like(sq_ref)

    ones = jnp.ones((_BLK, 1), jnp.float32)
    sum_ref[...] += jnp.dot(h, ones, preferred_element_type=jnp.float32)
    sq_ref[...] += jnp.dot(h * h, ones, preferred_element_type=jnp.float32)


def _stage2_body(h_ref, sum_ref, sq_ref, g1_ref, be_ref, w2_ref, b2_ref,
                 out_ref):
    inv_cnt = 1.0 / (_B * _N)
    mean = sum_ref[...] * inv_cnt
    var = sq_ref[...] * inv_cnt - mean * mean
    scale = g1_ref[...] * lax.rsqrt(var + 1e-5)
    shift = be_ref[...] - mean * scale
    g = jnp.maximum(h_ref[0].astype(jnp.float32) * scale + shift, 0.0)
    o = jnp.dot(w2_ref[...], g.astype(jnp.bfloat16),
                preferred_element_type=jnp.float32) + b2_ref[...]
    out_ref[0] = jnp.maximum(o, 0.0)


def kernel(xyz1, xyz2, points1, points2, W1, b1, gamma1, beta1, W2, b2):
    del b1
    x2t = jnp.transpose(xyz2, (0, 2, 1)) * (-2.0)
    w1a = W1[:, :_D2].astype(jnp.bfloat16)
    w1b = W1[:, _D2:].astype(jnp.bfloat16)

    h, hsum, hsq = pl.pallas_call(
        _stage1_body,
        grid=(_B, _NB),
        in_specs=[
            pl.BlockSpec((1, _S, 3), lambda b, n: (b, 0, 0)),
            pl.BlockSpec((1, 3, _BLK), lambda b, n: (b, 0, n)),
            pl.BlockSpec((1, _D2, _S), lambda b, n: (b, 0, 0)),
            pl.BlockSpec((1, _D1, _BLK), lambda b, n: (b, 0, n)),
            pl.BlockSpec((_M0, _D2), lambda b, n: (0, 0)),
            pl.BlockSpec((_M0, _D1), lambda b, n: (0, 0)),
        ],
        out_specs=[
            pl.BlockSpec((1, _M0, _BLK), lambda b, n: (b, 0, n)),
            pl.BlockSpec((_M0, 1), lambda b, n: (0, 0)),
            pl.BlockSpec((_M0, 1), lambda b, n: (0, 0)),
        ],
        out_shape=[
            jax.ShapeDtypeStruct((_B, _M0, _N), jnp.bfloat16),
            jax.ShapeDtypeStruct((_M0, 1), jnp.float32),
            jax.ShapeDtypeStruct((_M0, 1), jnp.float32),
        ],
    )(x2t, xyz1, points2, points1, w1a, w1b)

    out = pl.pallas_call(
        _stage2_body,
        grid=(_B, _NB2),
        in_specs=[
            pl.BlockSpec((1, _M0, _BLK2), lambda b, n: (b, 0, n)),
            pl.BlockSpec((_M0, 1), lambda b, n: (0, 0)),
            pl.BlockSpec((_M0, 1), lambda b, n: (0, 0)),
            pl.BlockSpec((_M0, 1), lambda b, n: (0, 0)),
            pl.BlockSpec((_M0, 1), lambda b, n: (0, 0)),
            pl.BlockSpec((_M1, _M0), lambda b, n: (0, 0)),
            pl.BlockSpec((_M1, 1), lambda b, n: (0, 0)),
        ],
        out_specs=pl.BlockSpec((1, _M1, _BLK2), lambda b, n: (b, 0, n)),
        out_shape=jax.ShapeDtypeStruct((_B, _M1, _N), jnp.float32),
    )(h, hsum, hsq, gamma1[:, None], beta1[:, None],
      W2.astype(jnp.bfloat16), b2[:, None])
    return out


# speedup vs baseline: 3.2871x; 1.0325x over previous
import jax
import jax.numpy as jnp
from jax import lax
from jax.experimental import pallas as pl

_B, _N, _S, _D1, _D2 = 16, 4096, 1024, 256, 512
_CIN = _D1 + _D2
_M0, _M1 = 512, 512
_BLK = 1024
_NB = _N // _BLK
_BLK2 = 1024
_NB2 = _N // _BLK2


def _stage0_body(p2_ref, w1a_ref, o_ref):
    o_ref[0] = jnp.dot(w1a_ref[...], p2_ref[0].astype(jnp.bfloat16),
                       preferred_element_type=jnp.float32).astype(jnp.bfloat16)


def _stage1_body(x2t_ref, x1_ref, wp2_ref, p1_ref, w1b_ref,
                 h_ref, sum_ref, sq_ref):
    x2t = x2t_ref[0]
    x1b = x1_ref[0]
    n2 = 0.25 * jnp.sum(x2t * x2t, axis=1, keepdims=True)
    n1 = jnp.sum(x1b * x1b, axis=0, keepdims=True)
    d = jnp.dot(x2t, x1b, preferred_element_type=jnp.float32) + (n2 + n1)

    m1 = jnp.min(d, axis=0, keepdims=True)
    m2 = jnp.min(jnp.where(d <= m1, jnp.float32(jnp.inf), d), axis=0,
                 keepdims=True)
    m3 = jnp.min(jnp.where(d <= m2, jnp.float32(jnp.inf), d), axis=0,
                 keepdims=True)
    inv_rs = 1.0 / (1.0 / (m1 + 1e-8) + 1.0 / (m2 + 1e-8) + 1.0 / (m3 + 1e-8))
    wgt = jnp.where(d <= m3, inv_rs / (d + 1e-8), 0.0).astype(jnp.bfloat16)

    h = jnp.dot(wp2_ref[0], wgt, preferred_element_type=jnp.float32)
    h = h + jnp.dot(w1b_ref[...], p1_ref[0].astype(jnp.bfloat16),
                    preferred_element_type=jnp.float32)
    h_ref[0] = h.astype(jnp.bfloat16)

    @pl.when((pl.program_id(0) == 0) & (pl.program_id(1) == 0))
    def _init():
        sum_ref[...] = jnp.zeros_like(sum_ref)
        sq_ref[...] = jnp.zeros_like(sq_ref)

    ones = jnp.ones((_BLK, 1), jnp.float32)
    sum_ref[...] += jnp.dot(h, ones, preferred_element_type=jnp.float32)
    sq_ref[...] += jnp.dot(h * h, ones, preferred_element_type=jnp.float32)


def _stage2_body(h_ref, sum_ref, sq_ref, g1_ref, be_ref, w2_ref, b2_ref,
                 out_ref):
    inv_cnt = 1.0 / (_B * _N)
    mean = sum_ref[...] * inv_cnt
    var = sq_ref[...] * inv_cnt - mean * mean
    scale = g1_ref[...] * lax.rsqrt(var + 1e-5)
    shift = be_ref[...] - mean * scale
    g = jnp.maximum(h_ref[0].astype(jnp.float32) * scale + shift, 0.0)
    o = jnp.dot(w2_ref[...], g.astype(jnp.bfloat16),
                preferred_element_type=jnp.float32) + b2_ref[...]
    out_ref[0] = jnp.maximum(o, 0.0)


def kernel(xyz1, xyz2, points1, points2, W1, b1, gamma1, beta1, W2, b2):
    del b1
    x2t = jnp.transpose(xyz2, (0, 2, 1)) * (-2.0)
    w1a = W1[:, :_D2].astype(jnp.bfloat16)
    w1b = W1[:, _D2:].astype(jnp.bfloat16)

    wp2 = pl.pallas_call(
        _stage0_body,
        grid=(_B,),
        in_specs=[
            pl.BlockSpec((1, _D2, _S), lambda b: (b, 0, 0)),
            pl.BlockSpec((_M0, _D2), lambda b: (0, 0)),
        ],
        out_specs=pl.BlockSpec((1, _M0, _S), lambda b: (b, 0, 0)),
        out_shape=jax.ShapeDtypeStruct((_B, _M0, _S), jnp.bfloat16),
    )(points2, w1a)

    h, hsum, hsq = pl.pallas_call(
        _stage1_body,
        grid=(_B, _NB),
        in_specs=[
            pl.BlockSpec((1, _S, 3), lambda b, n: (b, 0, 0)),
            pl.BlockSpec((1, 3, _BLK), lambda b, n: (b, 0, n)),
            pl.BlockSpec((1, _M0, _S), lambda b, n: (b, 0, 0)),
            pl.BlockSpec((1, _D1, _BLK), lambda b, n: (b, 0, n)),
            pl.BlockSpec((_M0, _D1), lambda b, n: (0, 0)),
        ],
        out_specs=[
            pl.BlockSpec((1, _M0, _BLK), lambda b, n: (b, 0, n)),
            pl.BlockSpec((_M0, 1), lambda b, n: (0, 0)),
            pl.BlockSpec((_M0, 1), lambda b, n: (0, 0)),
        ],
        out_shape=[
            jax.ShapeDtypeStruct((_B, _M0, _N), jnp.bfloat16),
            jax.ShapeDtypeStruct((_M0, 1), jnp.float32),
            jax.ShapeDtypeStruct((_M0, 1), jnp.float32),
        ],
    )(x2t, xyz1, wp2, points1, w1b)

    out = pl.pallas_call(
        _stage2_body,
        grid=(_B, _NB2),
        in_specs=[
            pl.BlockSpec((1, _M0, _BLK2), lambda b, n: (b, 0, n)),
            pl.BlockSpec((_M0, 1), lambda b, n: (0, 0)),
            pl.BlockSpec((_M0, 1), lambda b, n: (0, 0)),
            pl.BlockSpec((_M0, 1), lambda b, n: (0, 0)),
            pl.BlockSpec((_M0, 1), lambda b, n: (0, 0)),
            pl.BlockSpec((_M1, _M0), lambda b, n: (0, 0)),
            pl.BlockSpec((_M1, 1), lambda b, n: (0, 0)),
        ],
        out_specs=pl.BlockSpec((1, _M1, _BLK2), lambda b, n: (b, 0, n)),
        out_shape=jax.ShapeDtypeStruct((_B, _M1, _N), jnp.float32),
    )(h, hsum, hsq, gamma1[:, None], beta1[:, None],
      W2.astype(jnp.bfloat16), b2[:, None])
    return out


# speedup vs baseline: 3.3601x; 1.0222x over previous
import jax
import jax.numpy as jnp
from jax import lax
from jax.experimental import pallas as pl

_B, _N, _S, _D1, _D2 = 16, 4096, 1024, 256, 512
_CIN = _D1 + _D2
_M0, _M1 = 512, 512
_BLK = 1024
_NB = _N // _BLK
_BLK2 = 1024
_NB2 = _N // _BLK2


def _stage0_body(p2_ref, w1a_ref, x2_ref, o_ref, x24_ref):
    o_ref[0] = jnp.dot(w1a_ref[...], p2_ref[0].astype(jnp.bfloat16),
                       preferred_element_type=jnp.float32).astype(jnp.bfloat16)
    x2t = jnp.transpose(x2_ref[0], (1, 0))
    n2 = jnp.sum(x2t * x2t, axis=1, keepdims=True)
    x24_ref[0] = jnp.concatenate([x2t * (-2.0), n2], axis=1)


def _stage1_body(x24_ref, x1_ref, wp2_ref, p1_ref, w1b_ref,
                 h_ref, sum_ref, sq_ref):
    x24 = x24_ref[0]
    x1b = x1_ref[0]
    x1b4 = jnp.concatenate([x1b, jnp.ones((1, _BLK), jnp.float32)], axis=0)
    d0 = jnp.dot(x24, x1b4, preferred_element_type=jnp.float32)
    n1 = jnp.sum(x1b * x1b, axis=0, keepdims=True)

    m1 = jnp.min(d0, axis=0, keepdims=True)
    m2 = jnp.min(jnp.where(d0 <= m1, jnp.float32(jnp.inf), d0), axis=0,
                 keepdims=True)
    m3 = jnp.min(jnp.where(d0 <= m2, jnp.float32(jnp.inf), d0), axis=0,
                 keepdims=True)
    c = n1 + 1e-8
    inv_rs = 1.0 / (1.0 / (m1 + c) + 1.0 / (m2 + c) + 1.0 / (m3 + c))
    wgt = jnp.where(d0 <= m3, inv_rs / (d0 + c), 0.0).astype(jnp.bfloat16)

    h = jnp.dot(wp2_ref[0], wgt, preferred_element_type=jnp.float32)
    h = h + jnp.dot(w1b_ref[...], p1_ref[0].astype(jnp.bfloat16),
                    preferred_element_type=jnp.float32)
    h_ref[0] = h.astype(jnp.bfloat16)

    @pl.when((pl.program_id(0) == 0) & (pl.program_id(1) == 0))
    def _init():
        sum_ref[...] = jnp.zeros_like(sum_ref)
        sq_ref[...] = jnp.zeros_like(sq_ref)

    ones = jnp.ones((_BLK, 1), jnp.float32)
    sum_ref[...] += jnp.dot(h, ones, preferred_element_type=jnp.float32)
    sq_ref[...] += jnp.dot(h * h, ones, preferred_element_type=jnp.float32)


def _stage2_body(h_ref, sum_ref, sq_ref, g1_ref, be_ref, w2_ref, b2_ref,
                 out_ref):
    inv_cnt = 1.0 / (_B * _N)
    mean = sum_ref[...] * inv_cnt
    var = sq_ref[...] * inv_cnt - mean * mean
    scale = g1_ref[...] * lax.rsqrt(var + 1e-5)
    shift = be_ref[...] - mean * scale
    g = jnp.maximum(h_ref[0].astype(jnp.float32) * scale + shift, 0.0)
    o = jnp.dot(w2_ref[...], g.astype(jnp.bfloat16),
                preferred_element_type=jnp.float32) + b2_ref[...]
    out_ref[0] = jnp.maximum(o, 0.0)


def kernel(xyz1, xyz2, points1, points2, W1, b1, gamma1, beta1, W2, b2):
    del b1
    w1a = W1[:, :_D2].astype(jnp.bfloat16)
    w1b = W1[:, _D2:].astype(jnp.bfloat16)

    wp2, x24 = pl.pallas_call(
        _stage0_body,
        grid=(_B,),
        in_specs=[
            pl.BlockSpec((1, _D2, _S), lambda b: (b, 0, 0)),
            pl.BlockSpec((_M0, _D2), lambda b: (0, 0)),
            pl.BlockSpec((1, 3, _S), lambda b: (b, 0, 0)),
        ],
        out_specs=[
            pl.BlockSpec((1, _M0, _S), lambda b: (b, 0, 0)),
            pl.BlockSpec((1, _S, 4), lambda b: (b, 0, 0)),
        ],
        out_shape=[
            jax.ShapeDtypeStruct((_B, _M0, _S), jnp.bfloat16),
            jax.ShapeDtypeStruct((_B, _S, 4), jnp.float32),
        ],
    )(points2, w1a, xyz2)

    h, hsum, hsq = pl.pallas_call(
        _stage1_body,
        grid=(_B, _NB),
        in_specs=[
            pl.BlockSpec((1, _S, 4), lambda b, n: (b, 0, 0)),
            pl.BlockSpec((1, 3, _BLK), lambda b, n: (b, 0, n)),
            pl.BlockSpec((1, _M0, _S), lambda b, n: (b, 0, 0)),
            pl.BlockSpec((1, _D1, _BLK), lambda b, n: (b, 0, n)),
            pl.BlockSpec((_M0, _D1), lambda b, n: (0, 0)),
        ],
        out_specs=[
            pl.BlockSpec((1, _M0, _BLK), lambda b, n: (b, 0, n)),
            pl.BlockSpec((_M0, 1), lambda b, n: (0, 0)),
            pl.BlockSpec((_M0, 1), lambda b, n: (0, 0)),
        ],
        out_shape=[
            jax.ShapeDtypeStruct((_B, _M0, _N), jnp.bfloat16),
            jax.ShapeDtypeStruct((_M0, 1), jnp.float32),
            jax.ShapeDtypeStruct((_M0, 1), jnp.float32),
        ],
    )(x24, xyz1, wp2, points1, w1b)

    out = pl.pallas_call(
        _stage2_body,
        grid=(_B, _NB2),
        in_specs=[
            pl.BlockSpec((1, _M0, _BLK2), lambda b, n: (b, 0, n)),
            pl.BlockSpec((_M0, 1), lambda b, n: (0, 0)),
            pl.BlockSpec((_M0, 1), lambda b, n: (0, 0)),
            pl.BlockSpec((_M0, 1), lambda b, n: (0, 0)),
            pl.BlockSpec((_M0, 1), lambda b, n: (0, 0)),
            pl.BlockSpec((_M1, _M0), lambda b, n: (0, 0)),
            pl.BlockSpec((_M1, 1), lambda b, n: (0, 0)),
        ],
        out_specs=pl.BlockSpec((1, _M1, _BLK2), lambda b, n: (b, 0, n)),
        out_shape=jax.ShapeDtypeStruct((_B, _M1, _N), jnp.float32),
    )(h, hsum, hsq, gamma1[:, None], beta1[:, None],
      W2.astype(jnp.bfloat16), b2[:, None])
    return out
